# baseline (speedup 1.0000x reference)
import functools

import jax
import jax.numpy as jnp
from jax import lax
from jax.experimental import pallas as pl
from jax.experimental.pallas import tpu as pltpu
from jax.experimental.pallas import tpu_sc as plsc

N_FACTORS = 32
N_ROWS = 1000000
PITCH = 999936
TAIL = N_ROWS - PITCH
COLC = 249984
BATCH = 16384
LANES = 16
CHUNK = 128
FGRP = 8

_info = plsc.get_sparse_core_info()
_NC, _NS = _info.num_cores, _info.num_subcores
_NW = _NC * _NS
_BPW = BATCH // _NW
_NCHUNK = _BPW // CHUNK


def _detile_body(u_hbm, i_hbm, uo_hbm, io_hbm, sem):
    def fire(src, dst, k, j, s):
        f = pl.multiple_of(k * 4 + j, 8)
        pltpu.make_async_copy(
            src.at[f].at[pl.ds(0, PITCH)],
            dst.at[pl.ds(f * PITCH, PITCH)], sem.at[s]).start()

    def drain(src, dst, s):
        pltpu.make_async_copy(
            src.at[0].at[pl.ds(0, PITCH)],
            dst.at[pl.ds(0, PITCH)], sem.at[s]).wait()

    nr = N_FACTORS // 4

    def make_round(src, dst):
        def body(k, carry):
            par = (k % 2) * 4
            for j in range(4):
                fire(src, dst, k, j, par + j)

            @pl.when(k > 0)
            def _():
                for j in range(4):
                    drain(src, dst, 4 - par + j)
            return carry
        return body

    lax.fori_loop(0, nr, make_round(u_hbm, uo_hbm), 0, unroll=False)
    for j in range(4):
        drain(u_hbm, uo_hbm, ((nr - 1) % 2) * 4 + j)
    lax.fori_loop(0, nr, make_round(i_hbm, io_hbm), 0, unroll=False)
    for j in range(4):
        drain(i_hbm, io_hbm, ((nr - 1) % 2) * 4 + j)


@functools.partial(jax.jit, donate_argnums=())
def _detile(u_t, i_t):
    return pl.pallas_call(
        _detile_body,
        in_specs=[
            pl.BlockSpec(memory_space=pl.ANY),
            pl.BlockSpec(memory_space=pl.ANY),
        ],
        out_specs=[
            pl.BlockSpec(memory_space=pl.ANY),
            pl.BlockSpec(memory_space=pl.ANY),
        ],
        out_shape=[
            jax.ShapeDtypeStruct((N_FACTORS * PITCH,), jnp.float32),
            jax.ShapeDtypeStruct((N_FACTORS * PITCH,), jnp.float32),
        ],
        scratch_shapes=[
            pltpu.SemaphoreType.DMA((8,)),
        ],
    )(u_t, i_t)


def _sc_body(user_ref, item_ref, uf_ref, if_ref, ut_ref, it_ref, out_ref,
             idx_u, idx_i, cl_u, cl_i, tail_u, tail_v, u_buf, v_buf,
             out_v, sem):
    wid = lax.axis_index("s") * _NC + lax.axis_index("c")
    base = wid * _BPW

    pltpu.sync_copy(user_ref.at[pl.ds(base, _BPW)], idx_u)
    pltpu.sync_copy(item_ref.at[pl.ds(base, _BPW)], idx_i)
    pltpu.sync_copy(ut_ref, tail_u)
    pltpu.sync_copy(it_ref, tail_v)

    def clamp(c, carry):
        b0 = c * LANES
        cl_u[pl.ds(b0, LANES)] = jnp.minimum(idx_u[pl.ds(b0, LANES)],
                                             PITCH - 1)
        cl_i[pl.ds(b0, LANES)] = jnp.minimum(idx_i[pl.ds(b0, LANES)],
                                             PITCH - 1)
        return carry

    lax.fori_loop(0, _BPW // LANES, clamp, 0, unroll=False)

    def fgroup(g, carry):
        f0 = g * FGRP
        copies = []
        for df in range(FGRP):
            f = f0 + df
            row = pl.ds(f * PITCH, PITCH)
            for j in range(_NCHUNK):
                sl = pl.ds(j * CHUNK, CHUNK)
                copies.append(pltpu.async_copy(
                    uf_ref.at[row].at[cl_u.at[sl]], u_buf.at[f, sl], sem))
                copies.append(pltpu.async_copy(
                    if_ref.at[row].at[cl_i.at[sl]], v_buf.at[f, sl], sem))
        for cp in copies:
            cp.wait()
        return carry

    lax.fori_loop(0, N_FACTORS // FGRP, fgroup, 0, unroll=False)

    def chunk16(c, carry):
        b0 = c * LANES
        ru = idx_u[pl.ds(b0, LANES)]
        ri = idx_i[pl.ds(b0, LANES)]
        mu = ru >= PITCH
        mi = ri >= PITCH
        tu = jnp.maximum(ru - PITCH, 0)
        ti = jnp.maximum(ri - PITCH, 0)
        acc = jnp.zeros((LANES,), jnp.float32)
        for f in range(N_FACTORS):
            um = u_buf[f, pl.ds(b0, LANES)]
            vm = v_buf[f, pl.ds(b0, LANES)]
            ut = plsc.load_gather(tail_u, [tu + f * TAIL])
            vt = plsc.load_gather(tail_v, [ti + f * TAIL])
            uu = jnp.where(mu, ut, um)
            vv = jnp.where(mi, vt, vm)
            acc = acc + uu * vv
        out_v[pl.ds(b0, LANES)] = acc
        return carry

    lax.fori_loop(0, _BPW // LANES, chunk16, 0, unroll=False)

    pltpu.sync_copy(out_v, out_ref.at[pl.ds(base, _BPW)])


@jax.jit
def _sc_call(user, item, uf_flat, if_flat, u_tail, i_tail):
    mesh = plsc.VectorSubcoreMesh(core_axis_name="c", subcore_axis_name="s")
    return pl.kernel(
        _sc_body,
        mesh=mesh,
        compiler_params=pltpu.CompilerParams(
            needs_layout_passes=False, use_tc_tiling_on_sc=False),
        out_type=jax.ShapeDtypeStruct((BATCH,), jnp.float32),
        scratch_types=[
            pltpu.VMEM((_BPW,), jnp.int32),
            pltpu.VMEM((_BPW,), jnp.int32),
            pltpu.VMEM((_BPW,), jnp.int32),
            pltpu.VMEM((_BPW,), jnp.int32),
            pltpu.VMEM((N_FACTORS * TAIL,), jnp.float32),
            pltpu.VMEM((N_FACTORS * TAIL,), jnp.float32),
            pltpu.VMEM((N_FACTORS, _BPW), jnp.float32),
            pltpu.VMEM((N_FACTORS, _BPW), jnp.float32),
            pltpu.VMEM((_BPW,), jnp.float32),
            pltpu.SemaphoreType.DMA,
        ],
    )(user, item, uf_flat, if_flat, u_tail, i_tail)


def kernel(user, item, user_factors, item_factors):
    u_t = user_factors.T
    i_t = item_factors.T
    uf_flat, if_flat = _detile(u_t, i_t)
    u_tail = u_t[:, PITCH:].reshape(-1)
    i_tail = i_t[:, PITCH:].reshape(-1)
    return _sc_call(user.astype(jnp.int32), item.astype(jnp.int32),
                    uf_flat, if_flat, u_tail, i_tail)


# speedup vs baseline: 32.8267x; 32.8267x over previous
import functools

import jax
import jax.numpy as jnp
from jax import lax
from jax.experimental import pallas as pl
from jax.experimental.pallas import tpu as pltpu
from jax.experimental.pallas import tpu_sc as plsc

N_FACTORS = 32
N_ROWS = 1000000
PITCH = 999936
TAIL = N_ROWS - PITCH
COLC = 249984
BATCH = 16384
LANES = 16
CHUNK = 128
FGRP = 8

_info = plsc.get_sparse_core_info()
_NC, _NS = _info.num_cores, _info.num_subcores
_NW = _NC * _NS
_BPW = BATCH // _NW
_NCHUNK = _BPW // CHUNK


def _detile_body(u_hbm, i_hbm, uo_hbm, io_hbm,
                 b0, b1, b2, b3, in_sem, out_sem):
    bufs = [b0, b1, b2, b3]

    def make_round(src, dst):
        def body(k, carry):
            copies = []
            for j in range(4):
                f = pl.multiple_of(k * 4 + j, 8)
                copies.append(pltpu.make_async_copy(
                    src.at[f], bufs[j], in_sem.at[j]))
                copies[-1].start()
            outs = []
            for j in range(4):
                f = pl.multiple_of(k * 4 + j, 8)
                copies[j].wait()
                outs.append(pltpu.make_async_copy(
                    bufs[j].at[pl.ds(0, PITCH)],
                    dst.at[pl.ds(f * PITCH, PITCH)], out_sem.at[j]))
                outs[-1].start()
            for j in range(4):
                outs[j].wait()
            return carry
        return body

    lax.fori_loop(0, N_FACTORS // 4, make_round(u_hbm, uo_hbm), 0,
                  unroll=False)
    lax.fori_loop(0, N_FACTORS // 4, make_round(i_hbm, io_hbm), 0,
                  unroll=False)


@functools.partial(jax.jit, donate_argnums=())
def _detile(u_t, i_t):
    return pl.pallas_call(
        _detile_body,
        in_specs=[
            pl.BlockSpec(memory_space=pl.ANY),
            pl.BlockSpec(memory_space=pl.ANY),
        ],
        out_specs=[
            pl.BlockSpec(memory_space=pl.ANY),
            pl.BlockSpec(memory_space=pl.ANY),
        ],
        out_shape=[
            jax.ShapeDtypeStruct((N_FACTORS * PITCH,), jnp.float32),
            jax.ShapeDtypeStruct((N_FACTORS * PITCH,), jnp.float32),
        ],
        scratch_shapes=[
            pltpu.VMEM((N_ROWS,), jnp.float32),
            pltpu.VMEM((N_ROWS,), jnp.float32),
            pltpu.VMEM((N_ROWS,), jnp.float32),
            pltpu.VMEM((N_ROWS,), jnp.float32),
            pltpu.SemaphoreType.DMA((4,)),
            pltpu.SemaphoreType.DMA((4,)),
        ],
    )(u_t, i_t)


def _sc_body(user_ref, item_ref, uf_ref, if_ref, ut_ref, it_ref, out_ref,
             idx_u, idx_i, cl_u, cl_i, tail_u, tail_v, u_buf, v_buf,
             out_v, sem):
    wid = lax.axis_index("s") * _NC + lax.axis_index("c")
    base = wid * _BPW

    pltpu.sync_copy(user_ref.at[pl.ds(base, _BPW)], idx_u)
    pltpu.sync_copy(item_ref.at[pl.ds(base, _BPW)], idx_i)
    pltpu.sync_copy(ut_ref, tail_u)
    pltpu.sync_copy(it_ref, tail_v)

    def clamp(c, carry):
        b0 = c * LANES
        cl_u[pl.ds(b0, LANES)] = jnp.minimum(idx_u[pl.ds(b0, LANES)],
                                             PITCH - 1)
        cl_i[pl.ds(b0, LANES)] = jnp.minimum(idx_i[pl.ds(b0, LANES)],
                                             PITCH - 1)
        return carry

    lax.fori_loop(0, _BPW // LANES, clamp, 0, unroll=False)

    def fgroup(g, carry):
        f0 = g * FGRP
        copies = []
        for df in range(FGRP):
            f = f0 + df
            row = pl.ds(f * PITCH, PITCH)
            for j in range(_NCHUNK):
                sl = pl.ds(j * CHUNK, CHUNK)
                copies.append(pltpu.async_copy(
                    uf_ref.at[row].at[cl_u.at[sl]], u_buf.at[f, sl], sem))
                copies.append(pltpu.async_copy(
                    if_ref.at[row].at[cl_i.at[sl]], v_buf.at[f, sl], sem))
        for cp in copies:
            cp.wait()
        return carry

    lax.fori_loop(0, N_FACTORS // FGRP, fgroup, 0, unroll=False)

    def chunk16(c, carry):
        b0 = c * LANES
        ru = idx_u[pl.ds(b0, LANES)]
        ri = idx_i[pl.ds(b0, LANES)]
        mu = ru >= PITCH
        mi = ri >= PITCH
        tu = jnp.maximum(ru - PITCH, 0)
        ti = jnp.maximum(ri - PITCH, 0)
        acc = jnp.zeros((LANES,), jnp.float32)
        for f in range(N_FACTORS):
            um = u_buf[f, pl.ds(b0, LANES)]
            vm = v_buf[f, pl.ds(b0, LANES)]
            ut = plsc.load_gather(tail_u, [tu + f * TAIL])
            vt = plsc.load_gather(tail_v, [ti + f * TAIL])
            uu = jnp.where(mu, ut, um)
            vv = jnp.where(mi, vt, vm)
            acc = acc + uu * vv
        out_v[pl.ds(b0, LANES)] = acc
        return carry

    lax.fori_loop(0, _BPW // LANES, chunk16, 0, unroll=False)

    pltpu.sync_copy(out_v, out_ref.at[pl.ds(base, _BPW)])


@jax.jit
def _sc_call(user, item, uf_flat, if_flat, u_tail, i_tail):
    mesh = plsc.VectorSubcoreMesh(core_axis_name="c", subcore_axis_name="s")
    return pl.kernel(
        _sc_body,
        mesh=mesh,
        compiler_params=pltpu.CompilerParams(
            needs_layout_passes=False, use_tc_tiling_on_sc=False),
        out_type=jax.ShapeDtypeStruct((BATCH,), jnp.float32),
        scratch_types=[
            pltpu.VMEM((_BPW,), jnp.int32),
            pltpu.VMEM((_BPW,), jnp.int32),
            pltpu.VMEM((_BPW,), jnp.int32),
            pltpu.VMEM((_BPW,), jnp.int32),
            pltpu.VMEM((N_FACTORS * TAIL,), jnp.float32),
            pltpu.VMEM((N_FACTORS * TAIL,), jnp.float32),
            pltpu.VMEM((N_FACTORS, _BPW), jnp.float32),
            pltpu.VMEM((N_FACTORS, _BPW), jnp.float32),
            pltpu.VMEM((_BPW,), jnp.float32),
            pltpu.SemaphoreType.DMA,
        ],
    )(user, item, uf_flat, if_flat, u_tail, i_tail)


def kernel(user, item, user_factors, item_factors):
    u_t = user_factors.T
    i_t = item_factors.T
    uf_flat, if_flat = _detile(u_t, i_t)
    u_tail = u_t[:, PITCH:].reshape(-1)
    i_tail = i_t[:, PITCH:].reshape(-1)
    return _sc_call(user.astype(jnp.int32), item.astype(jnp.int32),
                    uf_flat, if_flat, u_tail, i_tail)
